# baseline (speedup 1.0000x reference)
import functools

import jax
import jax.numpy as jnp
from jax import lax
from jax.experimental import pallas as pl
from jax.experimental.pallas import tpu as pltpu
from jax.experimental.pallas import tpu_sc as plsc

_EMB = 32
_BATCH = 16384
_NC = 2
_NS = 16
_NW = _NC * _NS
_BW = _BATCH // _NW
_NBUF = 4


def _body(emb_t, idx_hbm, out, idx_v, col_v, gbuf, shared, sem, sem2):
    sid = lax.axis_index("s")
    wid = sid * _NC + lax.axis_index("c")
    base_i = wid * _BW
    pltpu.sync_copy(idx_hbm.at[pl.ds(base_i, _BW)], idx_v.at[pl.ds(0, _BW)])

    def ridx(i):
        return idx_v[pl.ds(i, 16)][0]

    def fire(i):
        r = ridx(i)
        j0 = pl.multiple_of((r >> 7) * 128, 128)
        pltpu.async_copy(
            emb_t.at[:, pl.ds(j0, 128)],
            shared.at[sid, lax.rem(i, _NBUF)],
            sem,
        )

    def drain_a(i):
        pltpu.make_async_copy(
            emb_t.at[:, pl.ds(0, 128)],
            shared.at[sid, lax.rem(i, _NBUF)],
            sem,
        ).wait()

    def fire_b(i):
        pltpu.async_copy(
            shared.at[sid, lax.rem(i, _NBUF)],
            gbuf.at[lax.rem(i, 2)],
            sem2,
        )

    def drain_b(i):
        pltpu.make_async_copy(
            emb_t.at[:, pl.ds(0, 128)], gbuf.at[lax.rem(i, 2)], sem2
        ).wait()

    iota16 = lax.broadcasted_iota(jnp.int32, (16,), 0)

    def extract(i):
        r = ridx(i)
        lane = jnp.full((16,), r & 127, jnp.int32)
        b = gbuf.at[lax.rem(i, 2)]
        pos = jnp.full((16,), i, jnp.int32)
        lo = plsc.load_gather(b, [iota16, lane])
        hi = plsc.load_gather(b, [iota16 + 16, lane])
        plsc.store_scatter(col_v, [iota16, pos], lo)
        plsc.store_scatter(col_v, [iota16 + 16, pos], hi)

    for i in range(_NBUF - 1):
        fire(i)
    drain_a(0)
    fire_b(0)

    def step(i, carry):

        @pl.when(i + (_NBUF - 1) < _BW)
        def _():
            fire(i + (_NBUF - 1))

        @pl.when(i + 1 < _BW)
        def _():
            drain_a(i + 1)
            fire_b(i + 1)

        drain_b(i)
        extract(i)
        return carry

    lax.fori_loop(0, _BW, step, 0)
    pltpu.sync_copy(col_v, out.at[:, pl.ds(base_i, _BW)])


@functools.partial(
    pl.kernel,
    mesh=plsc.VectorSubcoreMesh(core_axis_name="c", subcore_axis_name="s"),
    compiler_params=pltpu.CompilerParams(needs_layout_passes=False),
    out_type=jax.ShapeDtypeStruct((_EMB, _BATCH), jnp.float32),
    scratch_types=[
        pltpu.VMEM((_BW + 16,), jnp.int32),
        pltpu.VMEM((_EMB, _BW), jnp.float32),
        pltpu.VMEM((2, _EMB, 128), jnp.float32),
        pltpu.VMEM_SHARED((_NS, _NBUF, _EMB, 128), jnp.float32),
        pltpu.SemaphoreType.DMA,
        pltpu.SemaphoreType.DMA,
    ],
)
def _sc_gather(emb_t, idx_hbm, out, idx_v, col_v, gbuf, shared, sem, sem2):
    _body(emb_t, idx_hbm, out, idx_v, col_v, gbuf, shared, sem, sem2)


@jax.jit
def _impl(noise_idx, env_emb):
    out_t = _sc_gather(env_emb.T, noise_idx.astype(jnp.int32))
    return out_t.T[:, None, :]


def kernel(noise_idx, env_emb):
    return _impl(noise_idx, env_emb)


# speedup vs baseline: 1.7129x; 1.7129x over previous
import functools

import jax
import jax.numpy as jnp
from jax import lax
from jax.experimental import pallas as pl
from jax.experimental.pallas import tpu as pltpu
from jax.experimental.pallas import tpu_sc as plsc

_EMB = 32
_BATCH = 16384
_NC = 2
_NS = 16
_NW = _NC * _NS
_BW = _BATCH // _NW
_NBUF = 8


def _body(emb_t, idx_hbm, out, idx_v, col_v, gbuf, sem):
    wid = lax.axis_index("s") * _NC + lax.axis_index("c")
    base_i = wid * _BW
    pltpu.sync_copy(idx_hbm.at[pl.ds(base_i, _BW)], idx_v.at[pl.ds(0, _BW)])

    def ridx(i):
        return idx_v[pl.ds(i, 16)][0]

    def fire(i):
        r = ridx(i)
        j0 = pl.multiple_of((r >> 7) * 128, 128)
        pltpu.async_copy(
            emb_t.at[:, pl.ds(j0, 128)], gbuf.at[lax.rem(i, _NBUF)], sem
        )

    def drain(i):
        pltpu.make_async_copy(
            emb_t.at[:, pl.ds(0, 128)], gbuf.at[lax.rem(i, _NBUF)], sem
        ).wait()

    iota16 = lax.broadcasted_iota(jnp.int32, (16,), 0)

    def extract(i):
        r = ridx(i)
        lane = jnp.full((16,), r & 127, jnp.int32)
        b = gbuf.at[lax.rem(i, _NBUF)]
        pos = jnp.full((16,), i, jnp.int32)
        lo = plsc.load_gather(b, [iota16, lane])
        hi = plsc.load_gather(b, [iota16 + 16, lane])
        plsc.store_scatter(col_v, [iota16, pos], lo)
        plsc.store_scatter(col_v, [iota16 + 16, pos], hi)

    for i in range(_NBUF - 1):
        fire(i)

    def step(i, carry):

        @pl.when(i + (_NBUF - 1) < _BW)
        def _():
            fire(i + (_NBUF - 1))

        drain(i)
        extract(i)
        return carry

    lax.fori_loop(0, _BW, step, 0)
    pltpu.sync_copy(col_v, out.at[:, pl.ds(base_i, _BW)])


@functools.partial(
    pl.kernel,
    mesh=plsc.VectorSubcoreMesh(core_axis_name="c", subcore_axis_name="s"),
    compiler_params=pltpu.CompilerParams(needs_layout_passes=False),
    out_type=jax.ShapeDtypeStruct((_EMB, _BATCH), jnp.float32),
    scratch_types=[
        pltpu.VMEM((_BW + 16,), jnp.int32),
        pltpu.VMEM((_EMB, _BW), jnp.float32),
        pltpu.VMEM((_NBUF, _EMB, 128), jnp.float32),
        pltpu.SemaphoreType.DMA,
    ],
)
def _sc_gather(emb_t, idx_hbm, out, idx_v, col_v, gbuf, sem):
    _body(emb_t, idx_hbm, out, idx_v, col_v, gbuf, sem)


@jax.jit
def _impl(noise_idx, env_emb):
    out_t = _sc_gather(env_emb.T, noise_idx.astype(jnp.int32))
    return out_t.T[:, None, :]


def kernel(noise_idx, env_emb):
    return _impl(noise_idx, env_emb)


# speedup vs baseline: 1.7300x; 1.0099x over previous
import functools

import jax
import jax.numpy as jnp
from jax import lax
from jax.experimental import pallas as pl
from jax.experimental.pallas import tpu as pltpu
from jax.experimental.pallas import tpu_sc as plsc

_EMB = 32
_BATCH = 16384
_NC = 2
_NS = 16
_NW = _NC * _NS
_BW = _BATCH // _NW
_NBUF = 16


def _body(emb_t, idx_hbm, out, idx_v, col_v, gbuf, sem):
    wid = lax.axis_index("s") * _NC + lax.axis_index("c")
    base_i = wid * _BW
    pltpu.sync_copy(idx_hbm.at[pl.ds(base_i, _BW)], idx_v.at[pl.ds(0, _BW)])

    def ridx(i):
        return idx_v[pl.ds(i, 16)][0]

    def fire(i):
        r = ridx(i)
        j0 = pl.multiple_of((r >> 7) * 128, 128)
        pltpu.async_copy(
            emb_t.at[:, pl.ds(j0, 128)], gbuf.at[lax.rem(i, _NBUF)], sem
        )

    def drain(i):
        pltpu.make_async_copy(
            emb_t.at[:, pl.ds(0, 128)], gbuf.at[lax.rem(i, _NBUF)], sem
        ).wait()

    iota16 = lax.broadcasted_iota(jnp.int32, (16,), 0)

    def extract(i):
        r = ridx(i)
        lane = jnp.full((16,), r & 127, jnp.int32)
        b = gbuf.at[lax.rem(i, _NBUF)]
        pos = jnp.full((16,), i, jnp.int32)
        lo = plsc.load_gather(b, [iota16, lane])
        hi = plsc.load_gather(b, [iota16 + 16, lane])
        plsc.store_scatter(col_v, [iota16, pos], lo)
        plsc.store_scatter(col_v, [iota16 + 16, pos], hi)

    for i in range(_NBUF - 1):
        fire(i)

    def step(i, carry):

        @pl.when(i + (_NBUF - 1) < _BW)
        def _():
            fire(i + (_NBUF - 1))

        drain(i)
        extract(i)
        return carry

    lax.fori_loop(0, _BW, step, 0)
    pltpu.sync_copy(col_v, out.at[:, pl.ds(base_i, _BW)])


@functools.partial(
    pl.kernel,
    mesh=plsc.VectorSubcoreMesh(core_axis_name="c", subcore_axis_name="s"),
    compiler_params=pltpu.CompilerParams(needs_layout_passes=False),
    out_type=jax.ShapeDtypeStruct((_EMB, _BATCH), jnp.float32),
    scratch_types=[
        pltpu.VMEM((_BW + 16,), jnp.int32),
        pltpu.VMEM((_EMB, _BW), jnp.float32),
        pltpu.VMEM((_NBUF, _EMB, 128), jnp.float32),
        pltpu.SemaphoreType.DMA,
    ],
)
def _sc_gather(emb_t, idx_hbm, out, idx_v, col_v, gbuf, sem):
    _body(emb_t, idx_hbm, out, idx_v, col_v, gbuf, sem)


@jax.jit
def _impl(noise_idx, env_emb):
    out_t = _sc_gather(env_emb.T, noise_idx.astype(jnp.int32))
    return out_t.T[:, None, :]


def kernel(noise_idx, env_emb):
    return _impl(noise_idx, env_emb)
